# baseline (speedup 1.0000x reference)
import functools

import jax
import jax.numpy as jnp
import numpy as np
from jax import lax
from jax.experimental import pallas as pl
from jax.experimental.pallas import tpu as pltpu
from jax.experimental.pallas import tpu_sc as plsc

F32 = jnp.float32
BF16 = jnp.bfloat16

NUM_CORES = 2
NUM_SUBCORES = 16
LANES = 16


def _chi(half):
    p = np.empty(half, np.int32)
    hw = half // 2
    for t in range(hw):
        p[t] = 32 * (t // 16) + t % 16
        p[hw + t] = 32 * (t // 16) + 16 + t % 16
    return np.concatenate([p, p + half])


def _pack_rows(lo, hi):
    li = jax.lax.bitcast_convert_type(lo, jnp.int32)
    hi_i = jax.lax.bitcast_convert_type(hi, jnp.int32)
    lo16 = jax.lax.shift_right_logical(li + 0x8000, 16)
    hi16 = jax.lax.shift_right_logical(hi_i + 0x8000, 16)
    return lo16 | jax.lax.shift_left(hi16, 16)


def _pack_halves(hp, ob0, ob1):
    half = hp.shape[1] // 2
    hw = half // 2
    ob0[...] = _pack_rows(hp[:, :hw], hp[:, hw:half])
    ob1[...] = _pack_rows(hp[:, half:half + hw], hp[:, half + hw:])


def _atom_body(x_ref, w_ref, wp_ref, b_ref, bp_ref, jw_ref, jb_ref,
               o0, o1, ob0, ob1, jk_o):
    x = x_ref[...]
    h = jnp.dot(x, w_ref[...], preferred_element_type=F32) + b_ref[...]
    hp = jnp.dot(x.astype(BF16), wp_ref[...].astype(BF16),
                 preferred_element_type=F32) + bp_ref[...]
    half = h.shape[1] // 2
    o0[...] = h[:, :half]
    o1[...] = h[:, half:]
    _pack_halves(hp, ob0, ob1)
    jk_o[...] = (jnp.dot(h, jw_ref[...], preferred_element_type=F32)
                 + jb_ref[...])


def _atom_encode(x, W, Wp, b, bp, jW, jb, blk):
    n, d = x.shape
    h = W.shape[1]
    dout = jW.shape[1]
    assert n % blk == 0
    return pl.pallas_call(
        _atom_body,
        grid=(n // blk,),
        in_specs=[
            pl.BlockSpec((blk, d), lambda i: (i, 0)),
            pl.BlockSpec((d, h), lambda i: (0, 0)),
            pl.BlockSpec((d, h), lambda i: (0, 0)),
            pl.BlockSpec((1, h), lambda i: (0, 0)),
            pl.BlockSpec((1, h), lambda i: (0, 0)),
            pl.BlockSpec((h, dout), lambda i: (0, 0)),
            pl.BlockSpec((1, dout), lambda i: (0, 0)),
        ],
        out_specs=[
            pl.BlockSpec((blk, h // 2), lambda i: (i, 0)),
            pl.BlockSpec((blk, h // 2), lambda i: (i, 0)),
            pl.BlockSpec((blk, h // 4), lambda i: (i, 0)),
            pl.BlockSpec((blk, h // 4), lambda i: (i, 0)),
            pl.BlockSpec((blk, dout), lambda i: (i, 0)),
        ],
        out_shape=[
            jax.ShapeDtypeStruct((n, h // 2), F32),
            jax.ShapeDtypeStruct((n, h // 2), F32),
            jax.ShapeDtypeStruct((n, h // 4), jnp.int32),
            jax.ShapeDtypeStruct((n, h // 4), jnp.int32),
            jax.ShapeDtypeStruct((n, dout), F32),
        ],
    )(x, W, Wp, b.reshape(1, h), bp.reshape(1, h), jW, jb.reshape(1, dout))


def _bond_body(x_ref, wp_ref, bp_ref, ob0, ob1):
    hp = (jnp.dot(x_ref[...].astype(BF16), wp_ref[...].astype(BF16),
                  preferred_element_type=F32) + bp_ref[...])
    _pack_halves(hp, ob0, ob1)


def _bond_encode(x, Wp, bp, blk):
    n, d = x.shape
    h = Wp.shape[1]
    assert n % blk == 0
    return pl.pallas_call(
        _bond_body,
        grid=(n // blk,),
        in_specs=[
            pl.BlockSpec((blk, d), lambda i: (i, 0)),
            pl.BlockSpec((d, h), lambda i: (0, 0)),
            pl.BlockSpec((1, h), lambda i: (0, 0)),
        ],
        out_specs=[
            pl.BlockSpec((blk, h // 4), lambda i: (i, 0)),
            pl.BlockSpec((blk, h // 4), lambda i: (i, 0)),
        ],
        out_shape=[
            jax.ShapeDtypeStruct((n, h // 4), jnp.int32),
            jax.ShapeDtypeStruct((n, h // 4), jnp.int32),
        ],
    )(x, Wp, bp.reshape(1, h))


def _mlp_body(h0, h1, a0, a1, w1, b1, w2, b2, wp2, bp2, jw, jk_i,
              o0, o1, ob0, ob1, jk_o):
    z0 = h0[...] + a0[...]
    z1 = h1[...] + a1[...]
    t = jnp.dot(z0, w1[0], preferred_element_type=F32)
    t = t + jnp.dot(z1, w1[1], preferred_element_type=F32)
    t = jnp.maximum(t + b1[...], 0.0)
    u = jnp.maximum(jnp.dot(t, w2[...], preferred_element_type=F32)
                    + b2[...], 0.0)
    up = jnp.maximum(jnp.dot(t.astype(BF16), wp2[...].astype(BF16),
                             preferred_element_type=F32) + bp2[...], 0.0)
    half = u.shape[1] // 2
    o0[...] = u[:, :half]
    o1[...] = u[:, half:]
    _pack_halves(up, ob0, ob1)
    jk_o[...] = jk_i[...] + jnp.dot(u, jw[...], preferred_element_type=F32)


def _mlp_halves(h0, h1, agg, W1, b1, W2, b2, Wp2, bp2, jW, jk_in, blk):
    n, half = h0.shape
    hdim = W1.shape[1]
    dout = jW.shape[1]
    assert n % blk == 0
    nblk = n // blk
    w1r = W1.reshape(2, half, hdim)
    return pl.pallas_call(
        _mlp_body,
        grid=(nblk,),
        in_specs=[
            pl.BlockSpec((blk, half), lambda i: (i, 0)),
            pl.BlockSpec((blk, half), lambda i: (i, 0)),
            pl.BlockSpec((blk, half), lambda i: (i, 0)),
            pl.BlockSpec((blk, half), lambda i, _n=nblk: (i + _n, 0)),
            pl.BlockSpec((2, half, hdim), lambda i: (0, 0, 0)),
            pl.BlockSpec((1, hdim), lambda i: (0, 0)),
            pl.BlockSpec((hdim, hdim), lambda i: (0, 0)),
            pl.BlockSpec((1, hdim), lambda i: (0, 0)),
            pl.BlockSpec((hdim, hdim), lambda i: (0, 0)),
            pl.BlockSpec((1, hdim), lambda i: (0, 0)),
            pl.BlockSpec((hdim, dout), lambda i: (0, 0)),
            pl.BlockSpec((blk, dout), lambda i: (i, 0)),
        ],
        out_specs=[
            pl.BlockSpec((blk, half), lambda i: (i, 0)),
            pl.BlockSpec((blk, half), lambda i: (i, 0)),
            pl.BlockSpec((blk, half // 2), lambda i: (i, 0)),
            pl.BlockSpec((blk, half // 2), lambda i: (i, 0)),
            pl.BlockSpec((blk, dout), lambda i: (i, 0)),
        ],
        out_shape=[
            jax.ShapeDtypeStruct((n, half), F32),
            jax.ShapeDtypeStruct((n, half), F32),
            jax.ShapeDtypeStruct((n, half // 2), jnp.int32),
            jax.ShapeDtypeStruct((n, half // 2), jnp.int32),
            jax.ShapeDtypeStruct((n, dout), F32),
        ],
    )(h0, h1, agg, agg, w1r, b1.reshape(1, hdim), W2, b2.reshape(1, hdim),
      Wp2, bp2.reshape(1, hdim), jW, jk_in)


def _make_edge_fn(n_nodes, n_edges, half):
    ept = n_edges // NUM_SUBCORES
    assert n_edges % NUM_SUBCORES == 0
    ch = 48
    while ept % ch != 0:
        ch -= 8
    n_chunks = ept // ch
    assert n_chunks % 4 == 0
    rows_per_tile = n_nodes // NUM_SUBCORES
    assert n_nodes % NUM_SUBCORES == 0

    half_w = half // 2
    mesh = plsc.VectorSubcoreMesh(core_axis_name="c", subcore_axis_name="s")

    @functools.partial(
        pl.kernel,
        mesh=mesh,
        out_type=jax.ShapeDtypeStruct((NUM_CORES * n_nodes, half), F32),
        compiler_params=pltpu.CompilerParams(use_tc_tiling_on_sc=False),
        scratch_types=[
            pltpu.VMEM((4, ch), jnp.int32),
            pltpu.VMEM((4, ch), jnp.int32),
            pltpu.VMEM((2, ch, half_w), jnp.int32),
            pltpu.VMEM((2, ch, half_w), jnp.int32),
            pltpu.VMEM((2, ch, half), F32),
            pltpu.VMEM_SHARED((n_nodes, half), F32),
            pltpu.SemaphoreType.DMA,
            pltpu.SemaphoreType.DMA,
            pltpu.SemaphoreType.DMA,
            pltpu.SemaphoreType.DMA,
            pltpu.SemaphoreType.DMA,
            pltpu.SemaphoreType.DMA,
            pltpu.SemaphoreType.DMA,
            pltpu.SemaphoreType.DMA,
            pltpu.SemaphoreType.DMA,
            pltpu.SemaphoreType.DMA,
            pltpu.SemaphoreType.DMA,
            pltpu.SemaphoreType.DMA,
            pltpu.SemaphoreType.DMA,
            pltpu.SemaphoreType.DMA,
        ],
    )
    def edge_fn(hb0, hb1, eb0, eb1, src_h, dst_h, zeros, out,
                src_v, dst_v, rows_v, e_v, res_v, agg_sh,
                sg0, sg1, se0, se1, ss0, ss1,
                si0, si1, si2, si3, sd0, sd1, sd2, sd3):
        c = lax.axis_index("c")
        s = lax.axis_index("s")
        row_base = s * rows_per_tile
        tile_base = s * ept
        sgs = (sg0, sg1)
        ses = (se0, se1)
        sss = (ss0, ss1)
        sis = (si0, si1, si2, si3)
        sds = (sd0, sd1, sd2, sd3)

        pltpu.sync_copy(zeros.at[pl.ds(row_base, rows_per_tile)],
                        agg_sh.at[pl.ds(row_base, rows_per_tile)])
        plsc.subcore_barrier()

        def start_src(g, q):
            pltpu.async_copy(src_h.at[pl.ds(tile_base + g * ch, ch)],
                             src_v.at[q], sis[q])

        def wait_src(q):
            pltpu.make_async_copy(src_h.at[pl.ds(tile_base, ch)],
                                  src_v.at[q], sis[q]).wait()

        def start_dst(g, q):
            pltpu.async_copy(dst_h.at[pl.ds(tile_base + g * ch, ch)],
                             dst_v.at[q], sds[q])

        def wait_dst(q):
            pltpu.make_async_copy(dst_h.at[pl.ds(tile_base, ch)],
                                  dst_v.at[q], sds[q]).wait()

        def wait_s(p, q):
            pltpu.make_async_copy(res_v.at[p], agg_sh.at[dst_v.at[q]],
                                  sss[p]).wait()

        def compute(p):
            @plsc.parallel_loop(0, ch, 1, unroll=4)
            def _row(i):
                for k in range(half_w // LANES):
                    hw = rows_v[p, i, pl.ds(k * LANES, LANES)]
                    ew = e_v[p, i, pl.ds(k * LANES, LANES)]
                    af = (jax.lax.bitcast_convert_type(hw << 16, F32)
                          + jax.lax.bitcast_convert_type(ew << 16, F32))
                    bf = (jax.lax.bitcast_convert_type(hw & -65536, F32)
                          + jax.lax.bitcast_convert_type(ew & -65536, F32))
                    res_v[p, i, pl.ds(2 * k * LANES, LANES)] = (
                        jnp.maximum(af, 0.0))
                    res_v[p, i, pl.ds((2 * k + 1) * LANES, LANES)] = (
                        jnp.maximum(bf, 0.0))

        def run(h_ref, e_ref):
            def start_ge(g, p, q):
                pltpu.async_copy(h_ref.at[src_v.at[q]], rows_v.at[p], sgs[p])
                pltpu.async_copy(e_ref.at[pl.ds(tile_base + g * ch, ch)],
                                 e_v.at[p], ses[p])

            def wait_ge(p, q):
                pltpu.make_async_copy(h_ref.at[src_v.at[q]], rows_v.at[p],
                                      sgs[p]).wait()
                pltpu.make_async_copy(e_ref.at[pl.ds(tile_base, ch)],
                                      e_v.at[p], ses[p]).wait()

            for q in range(4):
                start_src(q, q)
            start_dst(0, 0)
            start_dst(1, 1)
            for g0 in range(2):
                wait_src(g0)
                start_ge(g0, g0, g0)

            def quad(gq, carry):
                g4 = gq * 4
                for q in (0, 1, 2, 3):
                    gg = g4 + q
                    p = q % 2
                    qn = (q + 2) % 4
                    wait_ge(p, q)

                    @pl.when(gg + 4 < n_chunks)
                    def _():
                        start_src(gg + 4, q)

                    @pl.when(gg >= 2)
                    def _():
                        wait_s(p, qn)

                    @pl.when(gg + 2 < n_chunks)
                    def _():
                        start_dst(gg + 2, qn)

                    compute(p)
                    wait_dst(q)
                    pltpu.async_copy(res_v.at[p], agg_sh.at[dst_v.at[q]],
                                     sss[p], add=True)

                    @pl.when(gg + 2 < n_chunks)
                    def _():
                        wait_src(qn)
                        start_ge(gg + 2, p, qn)
                return carry

            lax.fori_loop(0, n_chunks // 4, quad, 0)
            wait_s(0, 2)
            wait_s(1, 3)

        @pl.when(c == 0)
        def _():
            run(hb0, eb0)

        @pl.when(c == 1)
        def _():
            run(hb1, eb1)

        plsc.subcore_barrier()
        pltpu.sync_copy(agg_sh.at[pl.ds(row_base, rows_per_tile)],
                        out.at[pl.ds(c * n_nodes + row_base, rows_per_tile)])

    return edge_fn


def kernel(x, edge_index, edge_attr, atom_W, atom_b, bond_W, bond_b,
           conv_W1, conv_b1, conv_W2, conv_b2, jk_W, jk_b):
    n_nodes = x.shape[0]
    n_edges = edge_index.shape[1]
    hdim = atom_W.shape[1]
    half = hdim // 2
    n_layers = conv_W1.shape[0]

    n_pad = ((n_nodes + NUM_SUBCORES * 8 * 8 - 1)
             // (NUM_SUBCORES * 8 * 8)) * (NUM_SUBCORES * 8 * 8)
    x = jnp.pad(x, ((0, n_pad - n_nodes), (0, 0)))

    chi = _chi(half)
    atom_Wp = atom_W[:, chi]
    atom_bp = atom_b[chi]
    bond_Wp = bond_W[:, chi]
    bond_bp = bond_b[chi]
    conv_W2p = conv_W2[:, :, chi]
    conv_b2p = conv_b2[:, chi]

    src = edge_index[0]
    dst = edge_index[1]
    zeros = jnp.zeros((n_pad, half), dtype=F32)

    node_blk = n_pad // 16
    edge_blk = 2000

    dout = jk_W.shape[1]
    jkl = jk_W.reshape(n_layers + 1, hdim, dout)

    h0, h1, hbA, hbB, jk = _atom_encode(x, atom_W, atom_Wp, atom_b, atom_bp,
                                        jkl[0], jk_b, node_blk)
    ebA, ebB = _bond_encode(edge_attr, bond_Wp, bond_bp, edge_blk)

    edge_fn = _make_edge_fn(n_pad, n_edges, half)

    for l in range(n_layers):
        agg = edge_fn(hbA, hbB, ebA, ebB, src, dst, zeros)
        h0, h1, hbA, hbB, jk = _mlp_halves(h0, h1, agg, conv_W1[l],
                                           conv_b1[l], conv_W2[l], conv_b2[l],
                                           conv_W2p[l], conv_b2p[l],
                                           jkl[l + 1], jk, node_blk)

    return jk[:n_nodes]


# speedup vs baseline: 1.0016x; 1.0016x over previous
import functools

import jax
import jax.numpy as jnp
import numpy as np
from jax import lax
from jax.experimental import pallas as pl
from jax.experimental.pallas import tpu as pltpu
from jax.experimental.pallas import tpu_sc as plsc

F32 = jnp.float32
BF16 = jnp.bfloat16

NUM_CORES = 2
NUM_SUBCORES = 16
LANES = 16


def _chi(half):
    p = np.empty(half, np.int32)
    hw = half // 2
    for t in range(hw):
        p[t] = 32 * (t // 16) + t % 16
        p[hw + t] = 32 * (t // 16) + 16 + t % 16
    return np.concatenate([p, p + half])


def _pack_rows(lo, hi):
    li = jax.lax.bitcast_convert_type(lo, jnp.int32)
    hi_i = jax.lax.bitcast_convert_type(hi, jnp.int32)
    lo16 = jax.lax.shift_right_logical(li + 0x8000, 16)
    hi16 = jax.lax.shift_right_logical(hi_i + 0x8000, 16)
    return lo16 | jax.lax.shift_left(hi16, 16)


def _pack_halves(hp, ob0, ob1):
    half = hp.shape[1] // 2
    hw = half // 2
    ob0[...] = _pack_rows(hp[:, :hw], hp[:, hw:half])
    ob1[...] = _pack_rows(hp[:, half:half + hw], hp[:, half + hw:])


def _atom_body(x_ref, w_ref, wp_ref, b_ref, bp_ref, jw_ref, jb_ref,
               o0, o1, ob0, ob1, jk_o):
    x = x_ref[...]
    h = jnp.dot(x, w_ref[...], preferred_element_type=F32) + b_ref[...]
    hp = jnp.dot(x.astype(BF16), wp_ref[...].astype(BF16),
                 preferred_element_type=F32) + bp_ref[...]
    half = h.shape[1] // 2
    o0[...] = h[:, :half]
    o1[...] = h[:, half:]
    _pack_halves(hp, ob0, ob1)
    jk_o[...] = (jnp.dot(h, jw_ref[...], preferred_element_type=F32)
                 + jb_ref[...])


def _atom_encode(x, W, Wp, b, bp, jW, jb, blk):
    n, d = x.shape
    h = W.shape[1]
    dout = jW.shape[1]
    assert n % blk == 0
    return pl.pallas_call(
        _atom_body,
        grid=(n // blk,),
        in_specs=[
            pl.BlockSpec((blk, d), lambda i: (i, 0)),
            pl.BlockSpec((d, h), lambda i: (0, 0)),
            pl.BlockSpec((d, h), lambda i: (0, 0)),
            pl.BlockSpec((1, h), lambda i: (0, 0)),
            pl.BlockSpec((1, h), lambda i: (0, 0)),
            pl.BlockSpec((h, dout), lambda i: (0, 0)),
            pl.BlockSpec((1, dout), lambda i: (0, 0)),
        ],
        out_specs=[
            pl.BlockSpec((blk, h // 2), lambda i: (i, 0)),
            pl.BlockSpec((blk, h // 2), lambda i: (i, 0)),
            pl.BlockSpec((blk, h // 4), lambda i: (i, 0)),
            pl.BlockSpec((blk, h // 4), lambda i: (i, 0)),
            pl.BlockSpec((blk, dout), lambda i: (i, 0)),
        ],
        out_shape=[
            jax.ShapeDtypeStruct((n, h // 2), F32),
            jax.ShapeDtypeStruct((n, h // 2), F32),
            jax.ShapeDtypeStruct((n, h // 4), jnp.int32),
            jax.ShapeDtypeStruct((n, h // 4), jnp.int32),
            jax.ShapeDtypeStruct((n, dout), F32),
        ],
    )(x, W, Wp, b.reshape(1, h), bp.reshape(1, h), jW, jb.reshape(1, dout))


def _bond_body(x_ref, wp_ref, bp_ref, ob0, ob1):
    hp = (jnp.dot(x_ref[...].astype(BF16), wp_ref[...].astype(BF16),
                  preferred_element_type=F32) + bp_ref[...])
    _pack_halves(hp, ob0, ob1)


def _bond_encode(x, Wp, bp, blk):
    n, d = x.shape
    h = Wp.shape[1]
    assert n % blk == 0
    return pl.pallas_call(
        _bond_body,
        grid=(n // blk,),
        in_specs=[
            pl.BlockSpec((blk, d), lambda i: (i, 0)),
            pl.BlockSpec((d, h), lambda i: (0, 0)),
            pl.BlockSpec((1, h), lambda i: (0, 0)),
        ],
        out_specs=[
            pl.BlockSpec((blk, h // 4), lambda i: (i, 0)),
            pl.BlockSpec((blk, h // 4), lambda i: (i, 0)),
        ],
        out_shape=[
            jax.ShapeDtypeStruct((n, h // 4), jnp.int32),
            jax.ShapeDtypeStruct((n, h // 4), jnp.int32),
        ],
    )(x, Wp, bp.reshape(1, h))


def _mlp_body(h0, h1, a0, a1, w1, b1, w2, b2, wp2, bp2, jw, jk_i,
              o0, o1, ob0, ob1, jk_o):
    z0 = h0[...] + a0[...]
    z1 = h1[...] + a1[...]
    t = jnp.dot(z0, w1[0], preferred_element_type=F32)
    t = t + jnp.dot(z1, w1[1], preferred_element_type=F32)
    t = jnp.maximum(t + b1[...], 0.0)
    u = jnp.maximum(jnp.dot(t, w2[...], preferred_element_type=F32)
                    + b2[...], 0.0)
    up = jnp.maximum(jnp.dot(t.astype(BF16), wp2[...].astype(BF16),
                             preferred_element_type=F32) + bp2[...], 0.0)
    half = u.shape[1] // 2
    o0[...] = u[:, :half]
    o1[...] = u[:, half:]
    _pack_halves(up, ob0, ob1)
    jk_o[...] = jk_i[...] + jnp.dot(u, jw[...], preferred_element_type=F32)


def _mlp_halves(h0, h1, agg, W1, b1, W2, b2, Wp2, bp2, jW, jk_in, blk):
    n, half = h0.shape
    hdim = W1.shape[1]
    dout = jW.shape[1]
    assert n % blk == 0
    nblk = n // blk
    w1r = W1.reshape(2, half, hdim)
    return pl.pallas_call(
        _mlp_body,
        grid=(nblk,),
        in_specs=[
            pl.BlockSpec((blk, half), lambda i: (i, 0)),
            pl.BlockSpec((blk, half), lambda i: (i, 0)),
            pl.BlockSpec((blk, half), lambda i: (i, 0)),
            pl.BlockSpec((blk, half), lambda i, _n=nblk: (i + _n, 0)),
            pl.BlockSpec((2, half, hdim), lambda i: (0, 0, 0)),
            pl.BlockSpec((1, hdim), lambda i: (0, 0)),
            pl.BlockSpec((hdim, hdim), lambda i: (0, 0)),
            pl.BlockSpec((1, hdim), lambda i: (0, 0)),
            pl.BlockSpec((hdim, hdim), lambda i: (0, 0)),
            pl.BlockSpec((1, hdim), lambda i: (0, 0)),
            pl.BlockSpec((hdim, dout), lambda i: (0, 0)),
            pl.BlockSpec((blk, dout), lambda i: (i, 0)),
        ],
        out_specs=[
            pl.BlockSpec((blk, half), lambda i: (i, 0)),
            pl.BlockSpec((blk, half), lambda i: (i, 0)),
            pl.BlockSpec((blk, half // 2), lambda i: (i, 0)),
            pl.BlockSpec((blk, half // 2), lambda i: (i, 0)),
            pl.BlockSpec((blk, dout), lambda i: (i, 0)),
        ],
        out_shape=[
            jax.ShapeDtypeStruct((n, half), F32),
            jax.ShapeDtypeStruct((n, half), F32),
            jax.ShapeDtypeStruct((n, half // 2), jnp.int32),
            jax.ShapeDtypeStruct((n, half // 2), jnp.int32),
            jax.ShapeDtypeStruct((n, dout), F32),
        ],
    )(h0, h1, agg, agg, w1r, b1.reshape(1, hdim), W2, b2.reshape(1, hdim),
      Wp2, bp2.reshape(1, hdim), jW, jk_in)


def _make_edge_fn(n_nodes, n_edges, half):
    ept = n_edges // NUM_SUBCORES
    assert n_edges % NUM_SUBCORES == 0
    ch = 48
    while ept % ch != 0:
        ch -= 8
    n_chunks = ept // ch
    assert n_chunks % 4 == 0
    rows_per_tile = n_nodes // NUM_SUBCORES
    assert n_nodes % NUM_SUBCORES == 0

    half_w = half // 2
    mesh = plsc.VectorSubcoreMesh(core_axis_name="c", subcore_axis_name="s")

    @functools.partial(
        pl.kernel,
        mesh=mesh,
        out_type=jax.ShapeDtypeStruct((NUM_CORES * n_nodes, half), F32),
        compiler_params=pltpu.CompilerParams(use_tc_tiling_on_sc=False),
        scratch_types=[
            pltpu.VMEM((4, ch), jnp.int32),
            pltpu.VMEM((4, ch), jnp.int32),
            pltpu.VMEM((2, ch, half_w), jnp.int32),
            pltpu.VMEM((2, ch, half_w), jnp.int32),
            pltpu.VMEM((2, ch, half), F32),
            pltpu.VMEM_SHARED((n_nodes, half), F32),
            pltpu.SemaphoreType.DMA,
            pltpu.SemaphoreType.DMA,
            pltpu.SemaphoreType.DMA,
            pltpu.SemaphoreType.DMA,
            pltpu.SemaphoreType.DMA,
            pltpu.SemaphoreType.DMA,
            pltpu.SemaphoreType.DMA,
            pltpu.SemaphoreType.DMA,
            pltpu.SemaphoreType.DMA,
            pltpu.SemaphoreType.DMA,
            pltpu.SemaphoreType.DMA,
            pltpu.SemaphoreType.DMA,
            pltpu.SemaphoreType.DMA,
            pltpu.SemaphoreType.DMA,
        ],
    )
    def edge_fn(hb0, hb1, eb0, eb1, src_h, dst_h, zeros, out,
                src_v, dst_v, rows_v, e_v, res_v, agg_sh,
                sg0, sg1, se0, se1, ss0, ss1,
                si0, si1, si2, si3, sd0, sd1, sd2, sd3):
        c = lax.axis_index("c")
        s = lax.axis_index("s")
        row_base = s * rows_per_tile
        tile_base = s * ept
        sgs = (sg0, sg1)
        ses = (se0, se1)
        sss = (ss0, ss1)
        sis = (si0, si1, si2, si3)
        sds = (sd0, sd1, sd2, sd3)

        def start_src(g, q):
            pltpu.async_copy(src_h.at[pl.ds(tile_base + g * ch, ch)],
                             src_v.at[q], sis[q])

        def wait_src(q):
            pltpu.make_async_copy(src_h.at[pl.ds(tile_base, ch)],
                                  src_v.at[q], sis[q]).wait()

        def start_dst(g, q):
            pltpu.async_copy(dst_h.at[pl.ds(tile_base + g * ch, ch)],
                             dst_v.at[q], sds[q])

        def wait_dst(q):
            pltpu.make_async_copy(dst_h.at[pl.ds(tile_base, ch)],
                                  dst_v.at[q], sds[q]).wait()

        def wait_s(p, q):
            pltpu.make_async_copy(res_v.at[p], agg_sh.at[dst_v.at[q]],
                                  sss[p]).wait()

        def compute(p):
            @plsc.parallel_loop(0, ch, 1, unroll=4)
            def _row(i):
                for k in range(half_w // LANES):
                    hw = rows_v[p, i, pl.ds(k * LANES, LANES)]
                    ew = e_v[p, i, pl.ds(k * LANES, LANES)]
                    af = (jax.lax.bitcast_convert_type(hw << 16, F32)
                          + jax.lax.bitcast_convert_type(ew << 16, F32))
                    bf = (jax.lax.bitcast_convert_type(hw & -65536, F32)
                          + jax.lax.bitcast_convert_type(ew & -65536, F32))
                    res_v[p, i, pl.ds(2 * k * LANES, LANES)] = (
                        jnp.maximum(af, 0.0))
                    res_v[p, i, pl.ds((2 * k + 1) * LANES, LANES)] = (
                        jnp.maximum(bf, 0.0))

        def run(h_ref, e_ref):
            def start_ge(g, p, q):
                pltpu.async_copy(h_ref.at[src_v.at[q]], rows_v.at[p], sgs[p])
                pltpu.async_copy(e_ref.at[pl.ds(tile_base + g * ch, ch)],
                                 e_v.at[p], ses[p])

            def wait_ge(p, q):
                pltpu.make_async_copy(h_ref.at[src_v.at[q]], rows_v.at[p],
                                      sgs[p]).wait()
                pltpu.make_async_copy(e_ref.at[pl.ds(tile_base, ch)],
                                      e_v.at[p], ses[p]).wait()

            for q in range(4):
                start_src(q, q)
            start_dst(0, 0)
            start_dst(1, 1)
            pltpu.sync_copy(zeros.at[pl.ds(row_base, rows_per_tile)],
                            agg_sh.at[pl.ds(row_base, rows_per_tile)])
            for g0 in range(2):
                wait_src(g0)
                start_ge(g0, g0, g0)
            plsc.subcore_barrier()

            def quad(gq, carry):
                g4 = gq * 4
                for q in (0, 1, 2, 3):
                    gg = g4 + q
                    p = q % 2
                    qn = (q + 2) % 4
                    wait_ge(p, q)

                    @pl.when(gg + 4 < n_chunks)
                    def _():
                        start_src(gg + 4, q)

                    @pl.when(gg >= 2)
                    def _():
                        wait_s(p, qn)

                    @pl.when(gg + 2 < n_chunks)
                    def _():
                        start_dst(gg + 2, qn)

                    compute(p)
                    wait_dst(q)
                    pltpu.async_copy(res_v.at[p], agg_sh.at[dst_v.at[q]],
                                     sss[p], add=True)

                    @pl.when(gg + 2 < n_chunks)
                    def _():
                        wait_src(qn)
                        start_ge(gg + 2, p, qn)
                return carry

            lax.fori_loop(0, n_chunks // 4, quad, 0)
            wait_s(0, 2)
            wait_s(1, 3)

        @pl.when(c == 0)
        def _():
            run(hb0, eb0)

        @pl.when(c == 1)
        def _():
            run(hb1, eb1)

        plsc.subcore_barrier()
        pltpu.sync_copy(agg_sh.at[pl.ds(row_base, rows_per_tile)],
                        out.at[pl.ds(c * n_nodes + row_base, rows_per_tile)])

    return edge_fn


def kernel(x, edge_index, edge_attr, atom_W, atom_b, bond_W, bond_b,
           conv_W1, conv_b1, conv_W2, conv_b2, jk_W, jk_b):
    n_nodes = x.shape[0]
    n_edges = edge_index.shape[1]
    hdim = atom_W.shape[1]
    half = hdim // 2
    n_layers = conv_W1.shape[0]

    n_pad = ((n_nodes + NUM_SUBCORES * 8 * 8 - 1)
             // (NUM_SUBCORES * 8 * 8)) * (NUM_SUBCORES * 8 * 8)
    x = jnp.pad(x, ((0, n_pad - n_nodes), (0, 0)))

    chi = _chi(half)
    atom_Wp = atom_W[:, chi]
    atom_bp = atom_b[chi]
    bond_Wp = bond_W[:, chi]
    bond_bp = bond_b[chi]
    conv_W2p = conv_W2[:, :, chi]
    conv_b2p = conv_b2[:, chi]

    src = edge_index[0]
    dst = edge_index[1]
    zeros = jnp.zeros((n_pad, half), dtype=F32)

    node_blk = n_pad // 16
    edge_blk = 2000

    dout = jk_W.shape[1]
    jkl = jk_W.reshape(n_layers + 1, hdim, dout)

    h0, h1, hbA, hbB, jk = _atom_encode(x, atom_W, atom_Wp, atom_b, atom_bp,
                                        jkl[0], jk_b, node_blk)
    ebA, ebB = _bond_encode(edge_attr, bond_Wp, bond_bp, edge_blk)

    edge_fn = _make_edge_fn(n_pad, n_edges, half)

    for l in range(n_layers):
        agg = edge_fn(hbA, hbB, ebA, ebB, src, dst, zeros)
        h0, h1, hbA, hbB, jk = _mlp_halves(h0, h1, agg, conv_W1[l],
                                           conv_b1[l], conv_W2[l], conv_b2[l],
                                           conv_W2p[l], conv_b2p[l],
                                           jkl[l + 1], jk, node_blk)

    return jk[:n_nodes]
